# baseline (speedup 1.0000x reference)
import jax
import jax.numpy as jnp
from jax.experimental import pallas as pl
from jax.experimental.pallas import tpu as pltpu

_NUM_EXPERTS = 16
_TOKEN_BLOCK = 512


def _wsum_body(w_ref, s_ref):
    s_ref[0, 0, :] = jnp.sum(w_ref[0], axis=1)


def _moe_body(x_ref, gwt_ref, gb_ref, s_ref, b_ref, o_ref):
    x = x_ref[...]
    logits = jax.lax.dot_general(
        x, gwt_ref[...], (((1,), (0,)), ((), ())),
        preferred_element_type=jnp.float32,
    ) + gb_ref[...]
    g = jax.nn.softmax(logits, axis=-1)
    e_ids = jax.lax.broadcasted_iota(jnp.int32, g.shape, 1)
    i1 = jnp.argmax(g, axis=-1)
    oh1 = e_ids == i1[:, None]
    i2 = jnp.argmax(jnp.where(oh1, -1.0, g), axis=-1)
    oh2 = e_ids == i2[:, None]
    p = jnp.where(oh1 | oh2, g, 0.0)
    c = jax.lax.dot_general(
        p, s_ref[...], (((1,), (0,)), ((), ())),
        preferred_element_type=jnp.float32,
    )
    bias = jax.lax.dot_general(
        p, b_ref[...], (((1,), (0,)), ((), ())),
        preferred_element_type=jnp.float32,
    )
    o_ref[...] = x * c + bias


def kernel(x, gate_weight, gate_bias, experts_weights, experts_bias):
    n_tokens, d_in = x.shape
    n_exp, _, d_out = experts_weights.shape

    rows = 256
    s = pl.pallas_call(
        _wsum_body,
        grid=(n_exp, d_in // rows),
        in_specs=[pl.BlockSpec((1, rows, d_out), lambda e, i: (e, i, 0))],
        out_specs=pl.BlockSpec((1, 1, rows), lambda e, i: (e, 0, i)),
        out_shape=jax.ShapeDtypeStruct((n_exp, 1, d_in), jnp.float32),
    )(experts_weights)
    s = s.reshape(n_exp, d_in)

    blk = _TOKEN_BLOCK
    out = pl.pallas_call(
        _moe_body,
        grid=(n_tokens // blk,),
        in_specs=[
            pl.BlockSpec((blk, d_in), lambda i: (i, 0)),
            pl.BlockSpec((d_in, n_exp), lambda i: (0, 0)),
            pl.BlockSpec((1, n_exp), lambda i: (0, 0)),
            pl.BlockSpec((n_exp, d_in), lambda i: (0, 0)),
            pl.BlockSpec((n_exp, d_out), lambda i: (0, 0)),
        ],
        out_specs=pl.BlockSpec((blk, d_out), lambda i: (i, 0)),
        out_shape=jax.ShapeDtypeStruct((n_tokens, d_out), jnp.float32),
    )(x, gate_weight.T, gate_bias.reshape(1, n_exp), s, experts_bias)
    return out


# speedup vs baseline: 1.3994x; 1.3994x over previous
import jax
import jax.numpy as jnp
from jax.experimental import pallas as pl
from jax.experimental.pallas import tpu as pltpu

_NUM_EXPERTS = 16
_TOKEN_BLOCK = 512


def _wsum_body(w_ref, s_ref):
    s_ref[0, 0, :] = jnp.sum(w_ref[0], axis=1)


def _moe_body(x_ref, gwt_ref, gb_ref, sb_ref, o_ref):
    x = x_ref[...]
    logits = jax.lax.dot_general(
        x, gwt_ref[...], (((1,), (0,)), ((), ())),
        preferred_element_type=jnp.float32,
    ) + gb_ref[...]
    g = jax.nn.softmax(logits, axis=-1)
    e_ids = jax.lax.broadcasted_iota(jnp.int32, g.shape, 1)
    i1 = jnp.argmax(g, axis=-1)
    oh1 = e_ids == i1[:, None]
    i2 = jnp.argmax(jnp.where(oh1, -1.0, g), axis=-1)
    oh2 = e_ids == i2[:, None]
    p = jnp.where(oh1 | oh2, g, 0.0)
    d = x.shape[1]
    q = jax.lax.dot_general(
        p, sb_ref[...], (((1,), (0,)), ((), ())),
        preferred_element_type=jnp.float32,
    )
    o_ref[...] = x * q[:, :d] + q[:, d:]


def kernel(x, gate_weight, gate_bias, experts_weights, experts_bias):
    n_tokens, d_in = x.shape
    n_exp, _, d_out = experts_weights.shape

    s = pl.pallas_call(
        _wsum_body,
        grid=(n_exp,),
        in_specs=[pl.BlockSpec((1, d_in, d_out), lambda e: (e, 0, 0))],
        out_specs=pl.BlockSpec((1, 1, d_in), lambda e: (e, 0, 0)),
        out_shape=jax.ShapeDtypeStruct((n_exp, 1, d_in), jnp.float32),
    )(experts_weights)
    s = s.reshape(n_exp, d_in)

    sb = jnp.concatenate([s, experts_bias], axis=1)
    blk = _TOKEN_BLOCK
    out = pl.pallas_call(
        _moe_body,
        grid=(n_tokens // blk,),
        in_specs=[
            pl.BlockSpec((blk, d_in), lambda i: (i, 0)),
            pl.BlockSpec((d_in, n_exp), lambda i: (0, 0)),
            pl.BlockSpec((1, n_exp), lambda i: (0, 0)),
            pl.BlockSpec((n_exp, d_in + d_out), lambda i: (0, 0)),
        ],
        out_specs=pl.BlockSpec((blk, d_out), lambda i: (i, 0)),
        out_shape=jax.ShapeDtypeStruct((n_tokens, d_out), jnp.float32),
    )(x, gate_weight.T, gate_bias.reshape(1, n_exp), sb)
    return out


# speedup vs baseline: 1.5211x; 1.0870x over previous
import jax
import jax.numpy as jnp
from jax.experimental import pallas as pl
from jax.experimental.pallas import tpu as pltpu

_NUM_EXPERTS = 16
_TOKEN_BLOCK = 1024


def _wsum_body(w_ref, s_ref):
    s_ref[0, 0, :] = jnp.sum(w_ref[0], axis=1)


def _moe_body(x_ref, gwt_ref, gb_ref, sb_ref, o_ref):
    x = x_ref[...]
    logits = jax.lax.dot_general(
        x, gwt_ref[...], (((1,), (0,)), ((), ())),
        preferred_element_type=jnp.float32,
    ) + gb_ref[...]
    g = jax.nn.softmax(logits, axis=-1)
    e_ids = jax.lax.broadcasted_iota(jnp.int32, g.shape, 1)
    i1 = jnp.argmax(g, axis=-1)
    oh1 = e_ids == i1[:, None]
    i2 = jnp.argmax(jnp.where(oh1, -1.0, g), axis=-1)
    oh2 = e_ids == i2[:, None]
    p = jnp.where(oh1 | oh2, g, 0.0)
    d = x.shape[1]
    q = jax.lax.dot_general(
        p, sb_ref[...], (((1,), (0,)), ((), ())),
        preferred_element_type=jnp.float32,
    )
    o_ref[...] = x * q[:, :d] + q[:, d:]


def kernel(x, gate_weight, gate_bias, experts_weights, experts_bias):
    n_tokens, d_in = x.shape
    n_exp, _, d_out = experts_weights.shape

    s = pl.pallas_call(
        _wsum_body,
        grid=(n_exp,),
        in_specs=[pl.BlockSpec((1, d_in, d_out), lambda e: (e, 0, 0))],
        out_specs=pl.BlockSpec((1, 1, d_in), lambda e: (e, 0, 0)),
        out_shape=jax.ShapeDtypeStruct((n_exp, 1, d_in), jnp.float32),
    )(experts_weights)
    s = s.reshape(n_exp, d_in)

    sb = jnp.concatenate([s, experts_bias], axis=1)
    blk = _TOKEN_BLOCK
    out = pl.pallas_call(
        _moe_body,
        grid=(n_tokens // blk,),
        in_specs=[
            pl.BlockSpec((blk, d_in), lambda i: (i, 0)),
            pl.BlockSpec((d_in, n_exp), lambda i: (0, 0)),
            pl.BlockSpec((1, n_exp), lambda i: (0, 0)),
            pl.BlockSpec((n_exp, d_in + d_out), lambda i: (0, 0)),
        ],
        out_specs=pl.BlockSpec((blk, d_out), lambda i: (i, 0)),
        out_shape=jax.ShapeDtypeStruct((n_tokens, d_out), jnp.float32),
    )(x, gate_weight.T, gate_bias.reshape(1, n_exp), sb)
    return out


# speedup vs baseline: 1.5385x; 1.0114x over previous
import jax
import jax.numpy as jnp
from jax.experimental import pallas as pl
from jax.experimental.pallas import tpu as pltpu

_NUM_EXPERTS = 16
_TOKEN_BLOCK = 2048


def _wsum_body(w_ref, s_ref):
    s_ref[0, 0, :] = jnp.sum(w_ref[0], axis=1)


def _moe_body(x_ref, gwt_ref, gb_ref, sb_ref, o_ref):
    x = x_ref[...]
    logits = jax.lax.dot_general(
        x, gwt_ref[...], (((1,), (0,)), ((), ())),
        preferred_element_type=jnp.float32,
    ) + gb_ref[...]
    g = jax.nn.softmax(logits, axis=-1)
    e_ids = jax.lax.broadcasted_iota(jnp.int32, g.shape, 1)
    i1 = jnp.argmax(g, axis=-1)
    oh1 = e_ids == i1[:, None]
    i2 = jnp.argmax(jnp.where(oh1, -1.0, g), axis=-1)
    oh2 = e_ids == i2[:, None]
    p = jnp.where(oh1 | oh2, g, 0.0)
    d = x.shape[1]
    q = jax.lax.dot_general(
        p, sb_ref[...], (((1,), (0,)), ((), ())),
        preferred_element_type=jnp.float32,
    )
    o_ref[...] = x * q[:, :d] + q[:, d:]


def kernel(x, gate_weight, gate_bias, experts_weights, experts_bias):
    n_tokens, d_in = x.shape
    n_exp, _, d_out = experts_weights.shape

    s = pl.pallas_call(
        _wsum_body,
        grid=(n_exp,),
        in_specs=[pl.BlockSpec((1, d_in, d_out), lambda e: (e, 0, 0))],
        out_specs=pl.BlockSpec((1, 1, d_in), lambda e: (e, 0, 0)),
        out_shape=jax.ShapeDtypeStruct((n_exp, 1, d_in), jnp.float32),
    )(experts_weights)
    s = s.reshape(n_exp, d_in)

    sb = jnp.concatenate([s, experts_bias], axis=1)
    blk = _TOKEN_BLOCK
    out = pl.pallas_call(
        _moe_body,
        grid=(n_tokens // blk,),
        in_specs=[
            pl.BlockSpec((blk, d_in), lambda i: (i, 0)),
            pl.BlockSpec((d_in, n_exp), lambda i: (0, 0)),
            pl.BlockSpec((1, n_exp), lambda i: (0, 0)),
            pl.BlockSpec((n_exp, d_in + d_out), lambda i: (0, 0)),
        ],
        out_specs=pl.BlockSpec((blk, d_out), lambda i: (i, 0)),
        out_shape=jax.ShapeDtypeStruct((n_tokens, d_out), jnp.float32),
    )(x, gate_weight.T, gate_bias.reshape(1, n_exp), sb)
    return out
